# baseline (speedup 1.0000x reference)
import functools
import jax
import jax.numpy as jnp
from jax import lax
from jax.experimental import pallas as pl
from jax.experimental.pallas import tpu as pltpu
from jax.experimental.pallas import tpu_sc as plsc

D_MODEL = 64
HIST = 50
BATCH = 16384
NBB = BATCH // 128
NUM_WORKERS = 32
BB_PER_W = NBB // NUM_WORKERS
BB_PER_G = 2
GROUP = BB_PER_G * 128
N_GROUPS = HIST * (BB_PER_W // BB_PER_G)

_mesh = plsc.VectorSubcoreMesh(core_axis_name="c", subcore_axis_name="s")


@functools.partial(
    pl.kernel,
    mesh=_mesh,
    out_type=jax.ShapeDtypeStruct((HIST, 8, NBB, 8, 128), jnp.float32),
    scratch_types=[
        pltpu.VMEM((HIST, BB_PER_W * 128), jnp.int32),
        pltpu.VMEM((2, GROUP, D_MODEL), jnp.float32),
        pltpu.VMEM((2, 8, BB_PER_G, 8, 128), jnp.float32),
        pltpu.SemaphoreType.DMA,
        pltpu.SemaphoreType.DMA,
        pltpu.SemaphoreType.DMA,
        pltpu.SemaphoreType.DMA,
    ],
    compiler_params=pltpu.CompilerParams(
        use_tc_tiling_on_sc=False, needs_layout_passes=False),
)
def _gather_kernel(table_hbm, ids_hbm, out_hbm, idx_v, rows_v, trans_v,
                   gsem_a, gsem_b, osem_a, osem_b):
    wid = lax.axis_index("s") * 2 + lax.axis_index("c")
    bb0 = wid * BB_PER_W
    iota16 = lax.iota(jnp.int32, 16)
    bvecs = [iota16 + (j * 128 + bg * 16)
             for j in range(BB_PER_G) for bg in range(8)]

    pltpu.sync_copy(ids_hbm.at[:, pl.ds(bb0 * 128, BB_PER_W * 128)], idx_v)

    def gather_start(g, par, sem):
        h = g // 2
        half = g % 2
        pltpu.async_copy(
            table_hbm.at[idx_v.at[h, pl.ds(half * GROUP, GROUP)]],
            rows_v.at[par], sem)

    def gather_wait(par, sem):
        pltpu.make_async_copy(
            table_hbm.at[idx_v.at[0, pl.ds(0, GROUP)]], rows_v.at[par],
            sem).wait()

    def out_refs(g, par, dblk):
        h = g // 2
        half = g % 2
        return (trans_v.at[par, dblk],
                out_hbm.at[h, dblk, pl.ds(bb0 + half * BB_PER_G, BB_PER_G)])

    gather_start(0, 0, gsem_a)
    gather_start(1, 1, gsem_b)

    def body(k, _):
        for par, gsem, osem in ((0, gsem_a, osem_a), (1, gsem_b, osem_b)):
            g = k * 2 + par
            gather_wait(par, gsem)

            @pl.when(g >= 2)
            def _():
                for dblk in range(8):
                    src, dst = out_refs(g, par, dblk)
                    pltpu.make_async_copy(src, dst, osem).wait()

            @plsc.parallel_loop(0, 8, unroll=2)
            def _(dblk):
                for din in range(8):
                    dvec = jnp.full((16,), dblk * 8 + din, jnp.int32)
                    vals = [
                        plsc.load_gather(
                            rows_v.at[par], [bvecs[j * 8 + bg], dvec])
                        for j in range(BB_PER_G) for bg in range(8)
                    ]
                    for j in range(BB_PER_G):
                        for bg in range(8):
                            trans_v[par, dblk, j, din,
                                    pl.ds(bg * 16, 16)] = vals[j * 8 + bg]

            for dblk in range(8):
                src, dst = out_refs(g, par, dblk)
                pltpu.async_copy(src, dst, osem)

            @pl.when(g + 2 < N_GROUPS)
            def _():
                gather_start(g + 2, par, gsem)

        return 0

    lax.fori_loop(0, N_GROUPS // 2, body, 0, unroll=False)

    for par, osem in ((0, osem_a), (1, osem_b)):
        for dblk in range(8):
            src, dst = out_refs(N_GROUPS - 2 + par, par, dblk)
            pltpu.make_async_copy(src, dst, osem).wait()


def kernel(item_ids, item_matrix_weight):
    ids_t = item_ids.T.astype(jnp.int32)
    out5 = _gather_kernel(item_matrix_weight, ids_t)
    return out5.transpose(2, 4, 0, 1, 3).reshape(BATCH, HIST, D_MODEL)


# speedup vs baseline: 1.5122x; 1.5122x over previous
import functools
import jax
import jax.numpy as jnp
from jax import lax
from jax.experimental import pallas as pl
from jax.experimental.pallas import tpu as pltpu
from jax.experimental.pallas import tpu_sc as plsc

D_MODEL = 64
HIST = 50
BATCH = 16384
NBB = BATCH // 128
NUM_WORKERS = 32
BB_PER_W = NBB // NUM_WORKERS
BB_PER_G = 2
GROUP = BB_PER_G * 128
TRANS = BB_PER_G * 8 * 128
N_GROUPS = HIST * (BB_PER_W // BB_PER_G)

_mesh = plsc.VectorSubcoreMesh(core_axis_name="c", subcore_axis_name="s")


@functools.partial(
    pl.kernel,
    mesh=_mesh,
    out_type=jax.ShapeDtypeStruct((HIST, 8, NBB, 8, 128), jnp.float32),
    scratch_types=[
        pltpu.VMEM((HIST, BB_PER_W * 128), jnp.int32),
        pltpu.VMEM((2, GROUP, D_MODEL), jnp.float32),
        pltpu.VMEM((2, 8, BB_PER_G, 8, 128), jnp.float32),
        pltpu.SemaphoreType.DMA,
        pltpu.SemaphoreType.DMA,
        pltpu.SemaphoreType.DMA,
        pltpu.SemaphoreType.DMA,
    ],
    compiler_params=pltpu.CompilerParams(
        use_tc_tiling_on_sc=False, needs_layout_passes=False),
)
def _gather_kernel(table_hbm, ids_hbm, out_hbm, idx_v, rows_v, trans_v,
                   gsem_a, gsem_b, osem_a, osem_b):
    wid = lax.axis_index("s") * 2 + lax.axis_index("c")
    bb0 = wid * BB_PER_W
    iota16 = lax.iota(jnp.int32, 16)

    pltpu.sync_copy(ids_hbm.at[:, pl.ds(bb0 * 128, BB_PER_W * 128)], idx_v)

    def gather_start(g, par, sem):
        h = g // 2
        half = g % 2
        pltpu.async_copy(
            table_hbm.at[idx_v.at[h, pl.ds(half * GROUP, GROUP)]],
            rows_v.at[par], sem)

    def gather_wait(par, sem):
        pltpu.make_async_copy(
            table_hbm.at[idx_v.at[0, pl.ds(0, GROUP)]], rows_v.at[par],
            sem).wait()

    def out_refs(g, par, dblk):
        h = g // 2
        half = g % 2
        return (trans_v.at[par, dblk],
                out_hbm.at[h, dblk,
                           pl.ds(bb0 + half * BB_PER_G, BB_PER_G)])

    gather_start(0, 0, gsem_a)
    gather_start(1, 1, gsem_b)

    def body(kk, _):
        for par, gsem, osem in ((0, gsem_a, osem_a), (1, gsem_b, osem_b)):
            g = kk * 2 + par
            gather_wait(par, gsem)

            @pl.when(g >= 2)
            def _():
                for dblk in range(8):
                    src, dst = out_refs(g, par, dblk)
                    pltpu.make_async_copy(src, dst, osem).wait()

            @plsc.parallel_loop(0, 4, unroll=1)
            def _(i):
                d0 = i * 16

                def kbody(k, _c):
                    rot = (iota16 + k) % 16
                    dblkv = rot // 8 + i * 2
                    dinv = rot % 8
                    dvec = rot + d0
                    for j in range(BB_PER_G):
                        jv = jnp.full((16,), j, jnp.int32)
                        for bg in range(8):
                            bvec = iota16 + (j * 128 + bg * 16)
                            sbvec = iota16 + bg * 16
                            vals = plsc.load_gather(
                                rows_v.at[par], [bvec, dvec])
                            plsc.store_scatter(
                                trans_v.at[par], [dblkv, jv, dinv, sbvec],
                                vals)
                    return 0

                lax.fori_loop(0, 16, kbody, 0, unroll=False)

            for dblk in range(8):
                src, dst = out_refs(g, par, dblk)
                pltpu.async_copy(src, dst, osem)

            @pl.when(g + 2 < N_GROUPS)
            def _():
                gather_start(g + 2, par, gsem)

        return 0

    lax.fori_loop(0, N_GROUPS // 2, body, 0, unroll=False)

    for par, osem in ((0, osem_a), (1, osem_b)):
        for dblk in range(8):
            src, dst = out_refs(N_GROUPS - 2 + par, par, dblk)
            pltpu.make_async_copy(src, dst, osem).wait()


def kernel(item_ids, item_matrix_weight):
    ids_t = item_ids.T.astype(jnp.int32)
    out5 = _gather_kernel(item_matrix_weight, ids_t)
    return out5.transpose(2, 4, 0, 1, 3).reshape(BATCH, HIST, D_MODEL)


# speedup vs baseline: 1.5352x; 1.0152x over previous
import functools
import jax
import jax.numpy as jnp
from jax import lax
from jax.experimental import pallas as pl
from jax.experimental.pallas import tpu as pltpu
from jax.experimental.pallas import tpu_sc as plsc

D_MODEL = 64
HIST = 50
BATCH = 16384
NBB = BATCH // 128
NUM_WORKERS = 32
BB_PER_W = NBB // NUM_WORKERS
BB_PER_G = 2
GROUP = BB_PER_G * 128
TRANS = BB_PER_G * 8 * 128
N_GROUPS = HIST * (BB_PER_W // BB_PER_G)

_mesh = plsc.VectorSubcoreMesh(core_axis_name="c", subcore_axis_name="s")


@functools.partial(
    pl.kernel,
    mesh=_mesh,
    out_type=jax.ShapeDtypeStruct((HIST, 8, NBB, 8, 128), jnp.float32),
    scratch_types=[
        pltpu.VMEM((HIST, BB_PER_W * 128), jnp.int32),
        pltpu.VMEM((2, GROUP, D_MODEL), jnp.float32),
        pltpu.VMEM((2, 8, BB_PER_G, 8, 128), jnp.float32),
        pltpu.SemaphoreType.DMA,
        pltpu.SemaphoreType.DMA,
        pltpu.SemaphoreType.DMA,
        pltpu.SemaphoreType.DMA,
    ],
    compiler_params=pltpu.CompilerParams(
        use_tc_tiling_on_sc=False, needs_layout_passes=False),
)
def _gather_kernel(table_hbm, ids_hbm, out_hbm, idx_v, rows_v, trans_v,
                   gsem_a, gsem_b, osem_a, osem_b):
    wid = lax.axis_index("s") * 2 + lax.axis_index("c")
    bb0 = wid * BB_PER_W
    iota16 = lax.iota(jnp.int32, 16)

    pltpu.sync_copy(ids_hbm.at[:, pl.ds(bb0 * 128, BB_PER_W * 128)], idx_v)

    def gather_start(g, par, sem):
        h = g // 2
        half = g % 2
        pltpu.async_copy(
            table_hbm.at[idx_v.at[h, pl.ds(half * GROUP, GROUP)]],
            rows_v.at[par], sem)

    def gather_wait(par, sem):
        pltpu.make_async_copy(
            table_hbm.at[idx_v.at[0, pl.ds(0, GROUP)]], rows_v.at[par],
            sem).wait()

    def out_refs(g, par, dblk):
        h = g // 2
        half = g % 2
        return (trans_v.at[par, dblk],
                out_hbm.at[h, dblk,
                           pl.ds(bb0 + half * BB_PER_G, BB_PER_G)])

    gather_start(0, 0, gsem_a)
    gather_start(1, 1, gsem_b)

    def body(kk, _):
        for par, gsem, osem in ((0, gsem_a, osem_a), (1, gsem_b, osem_b)):
            g = kk * 2 + par
            gather_wait(par, gsem)

            @pl.when(g >= 2)
            def _():
                for dblk in range(8):
                    src, dst = out_refs(g, par, dblk)
                    pltpu.make_async_copy(src, dst, osem).wait()

            @plsc.parallel_loop(0, 4, unroll=1)
            def _(i):
                d0 = i * 16

                def kbody(k, _c):
                    rot = (iota16 + k) % 16
                    dblkv = rot // 8 + i * 2
                    dinv = rot % 8
                    dvec = rot + d0
                    for j in range(BB_PER_G):
                        jv = jnp.full((16,), j, jnp.int32)
                        for bg in range(8):
                            bvec = iota16 + (j * 128 + bg * 16)
                            sbvec = iota16 + bg * 16
                            vals = plsc.load_gather(
                                rows_v.at[par], [bvec, dvec])
                            plsc.store_scatter(
                                trans_v.at[par], [dblkv, jv, dinv, sbvec],
                                vals)
                    return 0

                lax.fori_loop(0, 16, kbody, 0, unroll=4)

            for dblk in range(8):
                src, dst = out_refs(g, par, dblk)
                pltpu.async_copy(src, dst, osem)

            @pl.when(g + 2 < N_GROUPS)
            def _():
                gather_start(g + 2, par, gsem)

        return 0

    lax.fori_loop(0, N_GROUPS // 2, body, 0, unroll=False)

    for par, osem in ((0, osem_a), (1, osem_b)):
        for dblk in range(8):
            src, dst = out_refs(N_GROUPS - 2 + par, par, dblk)
            pltpu.make_async_copy(src, dst, osem).wait()


def kernel(item_ids, item_matrix_weight):
    ids_t = item_ids.T.astype(jnp.int32)
    out5 = _gather_kernel(item_matrix_weight, ids_t)
    return out5.transpose(2, 4, 0, 1, 3).reshape(BATCH, HIST, D_MODEL)


# speedup vs baseline: 1.6246x; 1.0582x over previous
import functools
import jax
import jax.numpy as jnp
from jax import lax
from jax.experimental import pallas as pl
from jax.experimental.pallas import tpu as pltpu
from jax.experimental.pallas import tpu_sc as plsc

D_MODEL = 64
NUM_ROWS_PAD = 1000008
HIST = 50
BATCH = 16384
NBB = BATCH // 128
NUM_WORKERS = 32
BB_PER_W = NBB // NUM_WORKERS
BB_PER_G = 2
GROUP = BB_PER_G * 128
TRANS = BB_PER_G * 8 * 128
N_GROUPS = HIST * (BB_PER_W // BB_PER_G)

_mesh = plsc.VectorSubcoreMesh(core_axis_name="c", subcore_axis_name="s")


@functools.partial(
    pl.kernel,
    mesh=_mesh,
    out_type=jax.ShapeDtypeStruct((HIST, 8, NBB, 8, 128), jnp.float32),
    scratch_types=[
        pltpu.VMEM((HIST, BB_PER_W * 128), jnp.int32),
        pltpu.VMEM((2, GROUP), jnp.int32),
        pltpu.VMEM((2, GROUP, D_MODEL), jnp.float32),
        pltpu.VMEM((2, 8, BB_PER_G, 8, 128), jnp.float32),
        pltpu.SemaphoreType.DMA,
        pltpu.SemaphoreType.DMA,
        pltpu.SemaphoreType.DMA,
        pltpu.SemaphoreType.DMA,
    ],
    compiler_params=pltpu.CompilerParams(
        use_tc_tiling_on_sc=False, needs_layout_passes=False),
)
def _gather_kernel(table_hbm, ids_hbm, out_hbm, idx_v, idx2_v, rows_v,
                   trans_v, gsem_a, gsem_b, osem_a, osem_b):
    wid = lax.axis_index("s") * 2 + lax.axis_index("c")
    bb0 = wid * BB_PER_W
    iota16 = lax.iota(jnp.int32, 16)

    pltpu.sync_copy(ids_hbm.at[:, pl.ds(bb0 * 128, BB_PER_W * 128)], idx_v)

    def gather_start(g, par, sem):
        h = g // 2
        half = g % 2
        for t in range(GROUP // 16):
            v = idx_v[h, pl.ds(half * GROUP + t * 16, 16)]
            idx2_v[par, pl.ds(t * 16, 16)] = v * 2
        pltpu.async_copy(
            table_hbm.at[idx2_v.at[par]], rows_v.at[par], sem)

    def gather_wait(par, sem):
        pltpu.make_async_copy(
            table_hbm.at[idx2_v.at[par]], rows_v.at[par],
            sem).wait()

    def out_refs(g, par, dblk):
        h = g // 2
        half = g % 2
        return (trans_v.at[par, dblk],
                out_hbm.at[h, dblk,
                           pl.ds(bb0 + half * BB_PER_G, BB_PER_G)])

    gather_start(0, 0, gsem_a)
    gather_start(1, 1, gsem_b)

    def body(kk, _):
        for par, gsem, osem in ((0, gsem_a, osem_a), (1, gsem_b, osem_b)):
            g = kk * 2 + par
            gather_wait(par, gsem)

            @pl.when(g >= 2)
            def _():
                for dblk in range(8):
                    src, dst = out_refs(g, par, dblk)
                    pltpu.make_async_copy(src, dst, osem).wait()

            @plsc.parallel_loop(0, 4, unroll=1)
            def _(i):
                d0 = i * 16

                def kbody(k, _c):
                    rot = (iota16 + k) % 16
                    dblkv = rot // 8 + i * 2
                    dinv = rot % 8
                    dvec = rot + d0
                    for j in range(BB_PER_G):
                        jv = jnp.full((16,), j, jnp.int32)
                        for bg in range(8):
                            bvec = iota16 + (j * 128 + bg * 16)
                            sbvec = iota16 + bg * 16
                            vals = plsc.load_gather(
                                rows_v.at[par], [bvec, dvec])
                            plsc.store_scatter(
                                trans_v.at[par], [dblkv, jv, dinv, sbvec],
                                vals)
                    return 0

                lax.fori_loop(0, 16, kbody, 0, unroll=4)

            for dblk in range(8):
                src, dst = out_refs(g, par, dblk)
                pltpu.async_copy(src, dst, osem)

            @pl.when(g + 2 < N_GROUPS)
            def _():
                gather_start(g + 2, par, gsem)

        return 0

    lax.fori_loop(0, N_GROUPS // 2, body, 0, unroll=False)

    for par, osem in ((0, osem_a), (1, osem_b)):
        for dblk in range(8):
            src, dst = out_refs(N_GROUPS - 2 + par, par, dblk)
            pltpu.make_async_copy(src, dst, osem).wait()


def kernel(item_ids, item_matrix_weight):
    ids_t = item_ids.T.astype(jnp.int32)
    table_pad = jnp.pad(item_matrix_weight, ((0, 7), (0, D_MODEL)))
    out5 = _gather_kernel(table_pad.reshape(NUM_ROWS_PAD * 2, D_MODEL), ids_t)
    return out5.transpose(2, 4, 0, 1, 3).reshape(BATCH, HIST, D_MODEL)
